# baseline (speedup 1.0000x reference)
import functools

import jax
import jax.numpy as jnp
from jax import lax
from jax.experimental import pallas as pl
from jax.experimental.pallas import tpu as pltpu
from jax.experimental.pallas import tpu_sc as plsc

N = 49152
H = 64
NUM_MOD = 3
CHUNK = N // NUM_MOD

NC = 2
NS = 16
NW = NC * NS
PER_W = N // NW
CH = 128
NCH = PER_W // CH


def _mm_body(x_ref, w_ref, o_ref):
    o_ref[...] = jnp.dot(x_ref[...], w_ref[0], preferred_element_type=jnp.float32)


_RB = 2048
_NB = CHUNK // _RB


def _expert_matmul(x, wt):
    return pl.pallas_call(
        _mm_body,
        grid=(NUM_MOD, _NB),
        in_specs=[
            pl.BlockSpec((_RB, H), lambda e, b: (e * _NB + b, 0)),
            pl.BlockSpec((1, H, H), lambda e, b: (e, 0, 0)),
        ],
        out_specs=pl.BlockSpec((_RB, H), lambda e, b: (e * _NB + b, 0)),
        out_shape=jax.ShapeDtypeStruct((N, H), jnp.float32),
    )(x, wt)


_sc_mesh = plsc.VectorSubcoreMesh(core_axis_name="c", subcore_axis_name="s")


@functools.partial(
    pl.kernel,
    mesh=_sc_mesh,
    compiler_params=pltpu.CompilerParams(use_tc_tiling_on_sc=False),
    out_type=jax.ShapeDtypeStruct((N, H), jnp.float32),
    scratch_types=[
        pltpu.VMEM((NCH, CH), jnp.int32),
        pltpu.VMEM((NCH, CH), jnp.int32),
        pltpu.VMEM((PER_W, H), jnp.float32),
        pltpu.SemaphoreType.DMA,
        pltpu.SemaphoreType.DMA,
    ],
)
def _sc_permute(perm_hbm, inv_hbm, y_hbm, out_hbm, perm_v, idx_v, rows_v,
                sem_idx, sem_rows):
    wid = lax.axis_index("s") * NC + lax.axis_index("c")
    pltpu.sync_copy(perm_hbm.at[wid], perm_v)
    idx_copies = [
        pltpu.async_copy(inv_hbm.at[perm_v.at[j]], idx_v.at[j], sem_idx)
        for j in range(NCH)
    ]
    for c in idx_copies:
        c.wait()
    row_copies = [
        pltpu.async_copy(y_hbm.at[idx_v.at[j]], rows_v.at[pl.ds(j * CH, CH)],
                         sem_rows)
        for j in range(NCH)
    ]
    for c in row_copies:
        c.wait()
    pltpu.sync_copy(rows_v, out_hbm.at[pl.ds(wid * PER_W, PER_W)])


def kernel(x, permute_mapping, inv_permute_mapping, W0, W1, W2):
    wt = jnp.stack([W0.T, W1.T, W2.T])
    y = _expert_matmul(x, wt)
    perm3d = permute_mapping.reshape(NW, NCH, CH)
    return _sc_permute(perm3d, inv_permute_mapping, y)


# speedup vs baseline: 1.1847x; 1.1847x over previous
import functools

import jax
import jax.numpy as jnp
from jax import lax
from jax.experimental import pallas as pl
from jax.experimental.pallas import tpu as pltpu
from jax.experimental.pallas import tpu_sc as plsc

N = 49152
H = 64
NUM_MOD = 3
CHUNK = N // NUM_MOD

NC = 2
NS = 16
NW = NC * NS
PER_W = N // NW
CH = 128
NCH = PER_W // CH

NP = N // 2
PRB = 2048
_NB = (CHUNK // 2) // PRB


def _mm_body(x_ref, w_ref, o_ref):
    o_ref[...] = jnp.dot(x_ref[...], w_ref[0], preferred_element_type=jnp.float32)


def _expert_matmul(xp, wd):
    return pl.pallas_call(
        _mm_body,
        grid=(NUM_MOD, _NB),
        in_specs=[
            pl.BlockSpec((PRB, 2 * H), lambda e, b: (e * _NB + b, 0)),
            pl.BlockSpec((1, 2 * H, 2 * H), lambda e, b: (e, 0, 0)),
        ],
        out_specs=pl.BlockSpec((PRB, 2 * H), lambda e, b: (e * _NB + b, 0)),
        out_shape=jax.ShapeDtypeStruct((NP, 2 * H), jnp.float32),
    )(xp, wd)


_sc_mesh = plsc.VectorSubcoreMesh(core_axis_name="c", subcore_axis_name="s")


@functools.partial(
    pl.kernel,
    mesh=_sc_mesh,
    compiler_params=pltpu.CompilerParams(use_tc_tiling_on_sc=False),
    out_type=jax.ShapeDtypeStruct((N, H), jnp.float32),
    scratch_types=[
        pltpu.VMEM((NCH, CH), jnp.int32),
        pltpu.VMEM((NCH, CH), jnp.int32),
        pltpu.VMEM((PER_W, H), jnp.float32),
        pltpu.SemaphoreType.DMA,
        pltpu.SemaphoreType.DMA,
    ],
)
def _sc_permute(perm_hbm, inv_hbm, y_hbm, out_hbm, perm_v, idx_v, rows_v,
                sem_idx, sem_rows):
    wid = lax.axis_index("s") * NC + lax.axis_index("c")
    pltpu.sync_copy(perm_hbm.at[wid], perm_v)
    idx_copies = [
        pltpu.async_copy(inv_hbm.at[perm_v.at[j]], idx_v.at[j], sem_idx)
        for j in range(NCH)
    ]
    for c in idx_copies:
        c.wait()
    row_copies = [
        pltpu.async_copy(y_hbm.at[idx_v.at[j]], rows_v.at[pl.ds(j * CH, CH)],
                         sem_rows)
        for j in range(NCH)
    ]
    for c in row_copies:
        c.wait()
    pltpu.sync_copy(rows_v, out_hbm.at[pl.ds(wid * PER_W, PER_W)])


def _pair_blockdiag(w):
    z = jnp.zeros((H, H), jnp.float32)
    wt = w.T
    return jnp.concatenate(
        [jnp.concatenate([wt, z], axis=1), jnp.concatenate([z, wt], axis=1)],
        axis=0)


def kernel(x, permute_mapping, inv_permute_mapping, W0, W1, W2):
    wd = jnp.stack([_pair_blockdiag(W0), _pair_blockdiag(W1),
                    _pair_blockdiag(W2)])
    xp = x.reshape(NP, 2 * H)
    yp = _expert_matmul(xp, wd)
    y = yp.reshape(N, H)
    perm3d = permute_mapping.reshape(NW, NCH, CH)
    return _sc_permute(perm3d, inv_permute_mapping, y)
